# Initial kernel scaffold; baseline (speedup 1.0000x reference)
import jax
import jax.numpy as jnp
from jax.experimental import pallas as pl


def kernel(x, W_E):
    raise NotImplementedError("write your pallas kernel here")


# speedup vs baseline: 7.5561x; 7.5561x over previous
import functools

import jax
import jax.numpy as jnp
from jax import lax
from jax.experimental import pallas as pl
from jax.experimental.pallas import tpu as pltpu
from jax.experimental.pallas import tpu_sc as plsc


def _make_gather(V: int, D: int, N: int, NC: int, NS: int, C: int):
    NW = NC * NS
    assert N % (NW * C) == 0
    b_per_w = N // NW
    n_chunks = b_per_w // C
    mesh = plsc.VectorSubcoreMesh(
        core_axis_name="c", subcore_axis_name="s", num_cores=NC, num_subcores=NS
    )

    @functools.partial(
        pl.kernel,
        out_type=jax.ShapeDtypeStruct((N, D), jnp.float32),
        mesh=mesh,
        scratch_types=[
            pltpu.VMEM((n_chunks, C), jnp.int32),
            pltpu.VMEM((2, C, D), jnp.float32),
            pltpu.SemaphoreType.DMA,
        ],
    )
    def gather(wt_hbm, idx_hbm, out_hbm, idx_v, rows_v, gsem):
        wid = lax.axis_index("s") * NC + lax.axis_index("c")
        base = wid * b_per_w
        pltpu.sync_copy(idx_hbm.at[wid], idx_v.at[...])

        pltpu.async_copy(wt_hbm.at[idx_v.at[0]], rows_v.at[0], gsem).wait()

        def step(g, carry):
            @pl.when(g + 1 < n_chunks)
            def _():
                for s in range(2):
                    @pl.when(lax.rem(g + 1, 2) == s)
                    def _():
                        pltpu.async_copy(
                            wt_hbm.at[idx_v.at[g + 1]], rows_v.at[s], gsem
                        )

            for s in range(2):
                @pl.when(lax.rem(g, 2) == s)
                def _():
                    pltpu.sync_copy(
                        rows_v.at[s], out_hbm.at[pl.ds(base + g * C, C)]
                    )

            @pl.when(g + 1 < n_chunks)
            def _():
                pltpu.make_async_copy(
                    wt_hbm.at[idx_v.at[0]], rows_v.at[0], gsem
                ).wait()

            return carry

        lax.fori_loop(0, n_chunks, step, 0)

    return gather


def kernel(x, W_E):
    B, P = x.shape
    D, V = W_E.shape
    N = B * P
    WT = W_E.T
    info = plsc.get_sparse_core_info()
    NW = info.num_cores * info.num_subcores
    C = 128
    idx = x.reshape(NW, N // (NW * C), C).astype(jnp.int32)
    gather = _make_gather(V, D, N, info.num_cores, info.num_subcores, C=C)
    out = gather(WT, idx)
    return out.reshape(B, P, D)


# speedup vs baseline: 9.2713x; 1.2270x over previous
import functools

import jax
import jax.numpy as jnp
from jax import lax
from jax.experimental import pallas as pl
from jax.experimental.pallas import tpu as pltpu
from jax.experimental.pallas import tpu_sc as plsc


def _make_gather(V: int, D: int, N: int, NC: int, NS: int, C: int):
    NW = NC * NS
    assert N % (NW * C) == 0
    b_per_w = N // NW
    n_chunks = b_per_w // C
    mesh = plsc.VectorSubcoreMesh(
        core_axis_name="c", subcore_axis_name="s", num_cores=NC, num_subcores=NS
    )

    NBUF = 4
    assert n_chunks >= NBUF

    @functools.partial(
        pl.kernel,
        out_type=jax.ShapeDtypeStruct((N, D), jnp.float32),
        mesh=mesh,
        scratch_types=[
            pltpu.VMEM((n_chunks, C), jnp.int32),
            pltpu.VMEM((NBUF, C, D), jnp.float32),
            pltpu.SemaphoreType.DMA,
        ],
    )
    def gather(wt_hbm, idx_hbm, out_hbm, idx_v, rows_v, gsem):
        wid = lax.axis_index("s") * NC + lax.axis_index("c")
        base = wid * b_per_w
        pltpu.sync_copy(idx_hbm.at[wid], idx_v.at[...])

        for s in range(NBUF):
            pltpu.async_copy(wt_hbm.at[idx_v.at[s]], rows_v.at[s], gsem)

        def step(g, carry):
            pltpu.make_async_copy(
                wt_hbm.at[idx_v.at[0]], rows_v.at[0], gsem
            ).wait()
            for s in range(NBUF):
                @pl.when(lax.rem(g, NBUF) == s)
                def _():
                    pltpu.sync_copy(
                        rows_v.at[s], out_hbm.at[pl.ds(base + g * C, C)]
                    )

                    @pl.when(g + NBUF < n_chunks)
                    def _():
                        pltpu.async_copy(
                            wt_hbm.at[idx_v.at[g + NBUF]], rows_v.at[s], gsem
                        )

            return carry

        lax.fori_loop(0, n_chunks, step, 0)

    return gather


def kernel(x, W_E):
    B, P = x.shape
    D, V = W_E.shape
    N = B * P
    WT = W_E.T
    info = plsc.get_sparse_core_info()
    NW = info.num_cores * info.num_subcores
    C = 128
    idx = x.reshape(NW, N // (NW * C), C).astype(jnp.int32)
    gather = _make_gather(V, D, N, info.num_cores, info.num_subcores, C=C)
    out = gather(WT, idx)
    return out.reshape(B, P, D)


# speedup vs baseline: 9.3256x; 1.0059x over previous
import functools

import jax
import jax.numpy as jnp
from jax import lax
from jax.experimental import pallas as pl
from jax.experimental.pallas import tpu as pltpu
from jax.experimental.pallas import tpu_sc as plsc


def _make_gather(V: int, D: int, N: int, NC: int, NS: int, C: int):
    NW = NC * NS
    assert N % (NW * C) == 0
    b_per_w = N // NW
    n_chunks = b_per_w // C
    mesh = plsc.VectorSubcoreMesh(
        core_axis_name="c", subcore_axis_name="s", num_cores=NC, num_subcores=NS
    )

    NBUF = 6
    assert n_chunks >= NBUF

    @functools.partial(
        pl.kernel,
        out_type=jax.ShapeDtypeStruct((N, D), jnp.float32),
        mesh=mesh,
        scratch_types=[
            pltpu.VMEM((n_chunks, C), jnp.int32),
            pltpu.VMEM((NBUF, C, D), jnp.float32),
            pltpu.SemaphoreType.DMA,
            pltpu.SemaphoreType.DMA,
        ],
    )
    def gather(wt_hbm, idx_hbm, out_hbm, idx_v, rows_v, gsem, wsem):
        wid = lax.axis_index("s") * NC + lax.axis_index("c")
        base = wid * b_per_w
        pltpu.sync_copy(idx_hbm.at[wid], idx_v.at[...])

        for s in range(NBUF):
            pltpu.async_copy(wt_hbm.at[idx_v.at[s]], rows_v.at[s], gsem)

        def step(g, carry):
            pltpu.make_async_copy(
                wt_hbm.at[idx_v.at[0]], rows_v.at[0], gsem
            ).wait()
            for s in range(NBUF):
                @pl.when(lax.rem(g, NBUF) == s)
                def _():
                    pltpu.async_copy(
                        rows_v.at[s], out_hbm.at[pl.ds(base + g * C, C)], wsem
                    )

            @pl.when(jnp.logical_and(g >= 1, g - 1 + NBUF < n_chunks))
            def _():
                pltpu.make_async_copy(
                    rows_v.at[0], out_hbm.at[pl.ds(base, C)], wsem
                ).wait()
                for s in range(NBUF):
                    @pl.when(lax.rem(g - 1, NBUF) == s)
                    def _():
                        pltpu.async_copy(
                            wt_hbm.at[idx_v.at[g - 1 + NBUF]], rows_v.at[s], gsem
                        )

            return carry

        lax.fori_loop(0, n_chunks, step, 0)

        for _ in range(NBUF):
            pltpu.make_async_copy(
                rows_v.at[0], out_hbm.at[pl.ds(base, C)], wsem
            ).wait()

    return gather


def kernel(x, W_E):
    B, P = x.shape
    D, V = W_E.shape
    N = B * P
    WT = W_E.T
    info = plsc.get_sparse_core_info()
    NW = info.num_cores * info.num_subcores
    C = 128
    idx = x.reshape(NW, N // (NW * C), C).astype(jnp.int32)
    gather = _make_gather(V, D, N, info.num_cores, info.num_subcores, C=C)
    out = gather(WT, idx)
    return out.reshape(B, P, D)
